# Initial kernel scaffold; baseline (speedup 1.0000x reference)
import jax
import jax.numpy as jnp
from jax.experimental import pallas as pl


def kernel(x, pos_edge_index, neg_edge_index, W_pos, W_pos_cc, b_pos_cc, W_neg, W_neg_cc, b_neg_cc, W_pos_att, b_pos_att, W_neg_att, b_neg_att):
    raise NotImplementedError("write your pallas kernel here")


# speedup vs baseline: 3.1596x; 3.1596x over previous
import functools

import jax
import jax.numpy as jnp
from jax import lax
from jax.experimental import pallas as pl
from jax.experimental.pallas import tpu as pltpu
from jax.experimental.pallas import tpu_sc as plsc

N = 10000
E = 160000
D = 128
DK = 144
B = 64
NSUB = 16
NPAD = 10240
ROWS_PER_SUB = NPAD // NSUB
CHUNKS = E // B


def _att_proj_body(x_ref, w_ref, b_ref, o_ref):
    o_ref[...] = (
        jnp.dot(x_ref[...], w_ref[...], preferred_element_type=jnp.float32)
        + b_ref[...]
    )


def _att_proj(x, w_t, b):
    bm = 1000
    return pl.pallas_call(
        _att_proj_body,
        grid=(N // bm,),
        in_specs=[
            pl.BlockSpec((bm, D), lambda i: (i, 0)),
            pl.BlockSpec((D, 2 * D), lambda i: (0, 0)),
            pl.BlockSpec((1, 2 * D), lambda i: (0, 0)),
        ],
        out_specs=pl.BlockSpec((bm, 2 * D), lambda i: (i, 0)),
        out_shape=jax.ShapeDtypeStruct((N, 2 * D), jnp.float32),
    )(x, w_t, b)


def _sc_agg_body(x_hbm, att_hbm, rg_hbm, rs_hbm, l_hbm, outp_hbm, outn_hbm,
                 rg_v, rs_v, l_v, arow, xrow, msg, acc_sh, sem_a, sem_x):
    c = lax.axis_index("c")
    s = lax.axis_index("s")

    zvec = jnp.zeros((16,), jnp.float32)

    def zero_row(i, _):
        for k in range(DK // 16):
            msg[i, pl.ds(16 * k, 16)] = zvec
        return 0
    lax.fori_loop(0, B, zero_row, 0)

    row0 = s * ROWS_PER_SUB
    for j in range(ROWS_PER_SUB // B):
        pltpu.sync_copy(msg, acc_sh.at[pl.ds(row0 + j * B, B)])
    plsc.subcore_barrier()

    cnt_vec = jnp.where(lax.iota(jnp.int32, 16) == 0,
                        jnp.float32(1.0), jnp.float32(0.0))

    def cnt_row(i, _):
        msg[i, pl.ds(D, 16)] = cnt_vec
        return 0
    lax.fori_loop(0, B, cnt_row, 0)

    nbase = CHUNKS // NSUB
    n_chunks = nbase + jnp.where(s < CHUNKS - nbase * NSUB, 1, 0)

    def chunk_body(t, _):
        base = c * E + (s + NSUB * t) * B
        pltpu.sync_copy(rg_hbm.at[pl.ds(base, B)], rg_v)
        pltpu.sync_copy(l_hbm.at[pl.ds(base, B)], l_v)
        pltpu.sync_copy(rs_hbm.at[pl.ds(base, B)], rs_v)
        cp_a = pltpu.async_copy(att_hbm.at[rg_v], arow, sem_a)
        cp_x = pltpu.async_copy(x_hbm.at[l_v], xrow, sem_x)
        cp_a.wait()
        cp_x.wait()

        def edge_body(e, _):
            xv = [xrow[e, pl.ds(16 * k, 16)] for k in range(D // 16)]
            av = [arow[e, pl.ds(16 * k, 16)] for k in range(D // 16)]
            acc = av[0] * xv[0]
            for k in range(1, D // 16):
                acc = acc + av[k] * xv[k]
            att = jnp.sum(acc)
            for k in range(D // 16):
                msg[e, pl.ds(16 * k, 16)] = att * xv[k]
            return 0
        lax.fori_loop(0, B, edge_body, 0)

        pltpu.sync_copy(msg, acc_sh.at[rs_v], add=True)
        return 0
    lax.fori_loop(0, n_chunks, chunk_body, 0)

    plsc.subcore_barrier()

    @pl.when(c == 0)
    def _():
        pltpu.sync_copy(acc_sh.at[pl.ds(row0, ROWS_PER_SUB)],
                        outp_hbm.at[pl.ds(row0, ROWS_PER_SUB)])

    @pl.when(c == 1)
    def _():
        pltpu.sync_copy(acc_sh.at[pl.ds(row0, ROWS_PER_SUB)],
                        outn_hbm.at[pl.ds(row0, ROWS_PER_SUB)])


def _sc_agg(x, att_all, rg, rs, l_idx):
    mesh = plsc.VectorSubcoreMesh(core_axis_name="c", subcore_axis_name="s")
    f = pl.kernel(
        _sc_agg_body,
        out_type=[jax.ShapeDtypeStruct((NPAD, DK), jnp.float32),
                  jax.ShapeDtypeStruct((NPAD, DK), jnp.float32)],
        mesh=mesh,
        compiler_params=pltpu.CompilerParams(
            use_tc_tiling_on_sc=False, needs_layout_passes=False),
        scratch_types=[
            pltpu.VMEM((B,), jnp.int32),
            pltpu.VMEM((B,), jnp.int32),
            pltpu.VMEM((B,), jnp.int32),
            pltpu.VMEM((B, D), jnp.float32),
            pltpu.VMEM((B, D), jnp.float32),
            pltpu.VMEM((B, DK), jnp.float32),
            pltpu.VMEM_SHARED((NPAD, DK), jnp.float32),
            pltpu.SemaphoreType.DMA,
            pltpu.SemaphoreType.DMA,
        ],
    )
    return f(x, att_all, rg, rs, l_idx)


def _epilogue_body(sp_ref, sn_ref, x_ref, wp_ref, wpcc_ref, bp_ref,
                   wn_ref, wncc_ref, bn_ref, o_ref):
    sp = sp_ref[...]
    sn = sn_ref[...]
    xb = x_ref[...]
    mp = sp[:, :D] / jnp.maximum(sp[:, D:D + 1], 1.0)
    mn = sn[:, :D] / jnp.maximum(sn[:, D:D + 1], 1.0)
    op = (jnp.dot(mp, wp_ref[...], preferred_element_type=jnp.float32)
          + jnp.dot(xb, wpcc_ref[...], preferred_element_type=jnp.float32)
          + bp_ref[...])
    on = (jnp.dot(mn, wn_ref[...], preferred_element_type=jnp.float32)
          + jnp.dot(xb, wncc_ref[...], preferred_element_type=jnp.float32)
          + bn_ref[...])
    o_ref[...] = jnp.concatenate([op, on], axis=-1)


def _epilogue(sums_p, sums_n, x, wp_t, wpcc_t, bp, wn_t, wncc_t, bn):
    bm = 400
    do = 64
    return pl.pallas_call(
        _epilogue_body,
        grid=(N // bm,),
        in_specs=[
            pl.BlockSpec((bm, DK), lambda i: (i, 0)),
            pl.BlockSpec((bm, DK), lambda i: (i, 0)),
            pl.BlockSpec((bm, D), lambda i: (i, 0)),
            pl.BlockSpec((D, do), lambda i: (0, 0)),
            pl.BlockSpec((D, do), lambda i: (0, 0)),
            pl.BlockSpec((1, do), lambda i: (0, 0)),
            pl.BlockSpec((D, do), lambda i: (0, 0)),
            pl.BlockSpec((D, do), lambda i: (0, 0)),
            pl.BlockSpec((1, do), lambda i: (0, 0)),
        ],
        out_specs=pl.BlockSpec((bm, 2 * do), lambda i: (i, 0)),
        out_shape=jax.ShapeDtypeStruct((N, 2 * do), jnp.float32),
    )(sums_p, sums_n, x, wp_t, wpcc_t, bp, wn_t, wncc_t, bn)


def kernel(x, pos_edge_index, neg_edge_index, W_pos, W_pos_cc, b_pos_cc,
           W_neg, W_neg_cc, b_neg_cc, W_pos_att, b_pos_att, W_neg_att,
           b_neg_att):
    w_att_t = jnp.concatenate([W_pos_att.T, W_neg_att.T], axis=1)
    b_att = jnp.concatenate([b_pos_att, b_neg_att])[None, :]
    att_full = _att_proj(x, w_att_t, b_att)
    att_all = jnp.concatenate([att_full[:, :D], att_full[:, D:]], axis=0)

    rp, lp = pos_edge_index[0], pos_edge_index[1]
    rn, ln = neg_edge_index[0], neg_edge_index[1]
    rg = jnp.concatenate([rp, rn + N])
    rs = jnp.concatenate([rp, rn])
    l_idx = jnp.concatenate([lp, ln])

    sums_p, sums_n = _sc_agg(x, att_all, rg, rs, l_idx)

    return _epilogue(
        sums_p[:N], sums_n[:N], x, W_pos.T, W_pos_cc.T, b_pos_cc[None, :],
        W_neg.T, W_neg_cc.T, b_neg_cc[None, :])
